# baseline (speedup 1.0000x reference)
import jax
import jax.numpy as jnp
from jax.experimental import pallas as pl


def kernel(x, coords, edge_index, data_edge_index, Wa, ba, Wg, bg, Waf, baf, Wc1, bc1, Wm1, bm1, Wc2, bc2, Wf1, bf1, Wf2, bf2):
    N = x.shape[0]
    src, dst = edge_index[0], edge_index[1]
    p1 = x @ Wa[0, :512] + ba[0]
    p2 = x @ Wa[0, 512:]
    g1 = coords @ Wg[0, :4] + bg[0]
    g2 = coords @ Wg[0, 4:]
    hw1 = x @ Wc1.T
    x1 = jnp.maximum(p1[src] + p2[dst], 0.)
    x2 = jnp.maximum(g1[src] + g2[dst], 0.)
    ew = jnp.maximum(Waf[0, 0] * x1 + Waf[0, 1] * x2 + baf[0], 0.)
    deg = jnp.ones((N,), jnp.float32).at[dst].add(ew)
    dis = deg ** -0.5
    c1 = dis[src] * ew * dis[dst]
    acc1 = jnp.zeros((N, 256), jnp.float32).at[dst].add(c1[:, None] * hw1[src])
    h = jnp.maximum(acc1 + hw1 * (1.0 / deg)[:, None] + bc1, 0.)
    q1 = h @ Wm1[0, :256] + bm1[0]
    q2 = h @ Wm1[0, 256:]
    hw2 = h @ Wc2.T
    ea2 = jnp.maximum(q1[src] + q2[dst], 0.)
    deg2 = jnp.ones((N,), jnp.float32).at[dst].add(ea2)
    dis2 = deg2 ** -0.5
    c2 = dis2[src] * ea2 * dis2[dst]
    acc2 = jnp.zeros((N, 128), jnp.float32).at[dst].add(c2[:, None] * hw2[src])
    outF = acc2 + hw2 * (1.0 / deg2)[:, None]
    inp = outF[data_edge_index[0]] - outF[data_edge_index[1]]
    hid = jnp.maximum(inp @ Wf1.T + bf1, 0.)
    prob = jax.nn.sigmoid(hid @ Wf2.T + bf2)
    return prob


# speedup vs baseline: 10.0024x; 10.0024x over previous
import jax
import jax.numpy as jnp
from jax import lax
from jax.experimental import pallas as pl
from jax.experimental.pallas import tpu as pltpu
from jax.experimental.pallas import tpu_sc as plsc

F32 = jnp.float32
I32 = jnp.int32
NW = 32
L = 16


def _pad_edges(a, ep):
    e = a.shape[0]
    if e == ep:
        return a
    return jnp.concatenate([a, jnp.zeros((ep - e,), a.dtype)])


def _proj_body(x_ref, c_ref, wpa_ref, wgs_ref, b4_ref, o_ref):
    dn = (((1,), (1,)), ((), ()))
    pa = lax.dot_general(wpa_ref[...], x_ref[...], dn, preferred_element_type=F32)
    pg = lax.dot_general(wgs_ref[...], c_ref[...], dn, preferred_element_type=F32)
    o_ref[...] = jnp.concatenate([pa, pg], axis=0) + b4_ref[...]


def _hw1_body(x_ref, w_ref, o_ref):
    dn = (((1,), (1,)), ((), ()))
    hw = lax.dot_general(x_ref[...], w_ref[...], dn, preferred_element_type=F32)
    o_ref[0] = hw[:, :128]
    o_ref[1] = hw[:, 128:]


def _deg_body(degp_ref, dis_ref, inv_ref):
    deg = jnp.sum(degp_ref[...], axis=0, keepdims=True) + 1.0
    inv_ref[...] = 1.0 / deg
    dis_ref[...] = lax.rsqrt(deg)


def _stage_e_body(acc_ref, hwf_ref, inv_ref, bc1_ref, wc2a_ref, wc2b_ref,
                  wqa_ref, wqb_ref, bq_ref, hw2_ref, q_ref):
    dn = (((1,), (1,)), ((), ()))
    dn0 = (((1,), (0,)), ((), ()))
    inv = inv_ref[...]
    h0 = jnp.maximum(acc_ref[0] + hwf_ref[0] * inv + bc1_ref[0], 0.0)
    h1 = jnp.maximum(acc_ref[1] + hwf_ref[1] * inv + bc1_ref[1], 0.0)
    hw2_ref[...] = (lax.dot_general(h0, wc2a_ref[...], dn, preferred_element_type=F32)
                    + lax.dot_general(h1, wc2b_ref[...], dn, preferred_element_type=F32))
    q_ref[...] = (lax.dot_general(h0, wqa_ref[...], dn0, preferred_element_type=F32)
                  + lax.dot_general(h1, wqb_ref[...], dn0, preferred_element_type=F32)
                  + bq_ref[...])


def _combine_body(accp_ref, hw2_ref, inv2_ref, o_ref):
    o_ref[...] = accp_ref[0] + accp_ref[1] + hw2_ref[...] * inv2_ref[...]


def _mlp_body(inp_ref, wf1_ref, bf1_ref, wf2_ref, bf2_ref, o_ref):
    dn = (((1,), (1,)), ((), ()))
    hid = jnp.maximum(
        lax.dot_general(inp_ref[...], wf1_ref[...], dn, preferred_element_type=F32)
        + bf1_ref[...], 0.0)
    logits = jnp.sum(hid * wf2_ref[...], axis=1, keepdims=True) + bf2_ref[0, 0]
    o_ref[...] = jax.nn.sigmoid(logits)


def _sc_mesh():
    return plsc.VectorSubcoreMesh(core_axis_name="c", subcore_axis_name="s",
                                  num_cores=2, num_subcores=16)


_SC_PARAMS = pltpu.CompilerParams(needs_layout_passes=False)


def _make_affinity(N, E, EP):
    EPW = EP // NW

    def body(src_h, dst_h, pg_h, cst_h, ew_h, degp_h,
             sv, dv, p1v, p2v, g1v, g2v, degv, ewv, cstv):
        c = lax.axis_index("c")
        s = lax.axis_index("s")
        wid = s * 2 + c
        base = wid * EPW
        pltpu.sync_copy(src_h.at[pl.ds(base, EPW)], sv)
        pltpu.sync_copy(dst_h.at[pl.ds(base, EPW)], dv)
        pltpu.sync_copy(pg_h.at[pl.ds(0, N)], p1v)
        pltpu.sync_copy(pg_h.at[pl.ds(N, N)], p2v)
        pltpu.sync_copy(pg_h.at[pl.ds(2 * N, N)], g1v)
        pltpu.sync_copy(pg_h.at[pl.ds(3 * N, N)], g2v)
        pltpu.sync_copy(cst_h, cstv)
        w0 = cstv[pl.ds(0, L)]
        w1 = cstv[pl.ds(L, L)]
        bafv = cstv[pl.ds(2 * L, L)]

        def zero(i, carry):
            degv[pl.ds(i * L, L)] = jnp.full((L,), 0.0, F32)
            return carry
        lax.fori_loop(0, N // L, zero, 0)

        iot = lax.iota(I32, L)

        def step(i, carry):
            off = i * L
            s16 = sv[pl.ds(off, L)]
            d16 = dv[pl.ds(off, L)]
            x1 = jnp.maximum(plsc.load_gather(p1v, [s16]) + plsc.load_gather(p2v, [d16]), 0.0)
            x2 = jnp.maximum(plsc.load_gather(g1v, [s16]) + plsc.load_gather(g2v, [d16]), 0.0)
            ew16 = jnp.maximum(w0 * x1 + w1 * x2 + bafv, 0.0)
            ew16 = jnp.where(base + off + iot < E, ew16, 0.0)
            ewv[pl.ds(off, L)] = ew16
            plsc.addupdate_scatter(degv, [d16], ew16)
            return carry
        lax.fori_loop(0, EPW // L, step, 0)

        pltpu.sync_copy(ewv, ew_h.at[pl.ds(base, EPW)])
        pltpu.sync_copy(degv, degp_h.at[pl.ds(wid * N, N)])

    return pl.kernel(
        body,
        out_type=(jax.ShapeDtypeStruct((EP,), F32),
                  jax.ShapeDtypeStruct((NW * N,), F32)),
        mesh=_sc_mesh(),
        scratch_types=[
            pltpu.VMEM((EPW,), I32), pltpu.VMEM((EPW,), I32),
            pltpu.VMEM((N,), F32), pltpu.VMEM((N,), F32),
            pltpu.VMEM((N,), F32), pltpu.VMEM((N,), F32),
            pltpu.VMEM((N,), F32), pltpu.VMEM((EPW,), F32),
            pltpu.VMEM((3 * L,), F32),
        ],
        compiler_params=_SC_PARAMS,
        name="sc_affinity",
    )


def _make_affinity2(N, NP, E, EP):
    EPW = EP // NW

    def body(src_h, dst_h, qf_h, ea_h, degp_h, sv, dv, qv, degv, eav):
        c = lax.axis_index("c")
        s = lax.axis_index("s")
        wid = s * 2 + c
        base = wid * EPW
        pltpu.sync_copy(src_h.at[pl.ds(base, EPW)], sv)
        pltpu.sync_copy(dst_h.at[pl.ds(base, EPW)], dv)
        pltpu.sync_copy(qf_h, qv)

        def zero(i, carry):
            degv[pl.ds(i * L, L)] = jnp.full((L,), 0.0, F32)
            return carry
        lax.fori_loop(0, N // L, zero, 0)

        iot = lax.iota(I32, L)

        def step(i, carry):
            off = i * L
            s16 = sv[pl.ds(off, L)]
            d16 = dv[pl.ds(off, L)]
            ea16 = jnp.maximum(
                plsc.load_gather(qv, [s16 * 2]) + plsc.load_gather(qv, [d16 * 2 + 1]), 0.0)
            ea16 = jnp.where(base + off + iot < E, ea16, 0.0)
            eav[pl.ds(off, L)] = ea16
            plsc.addupdate_scatter(degv, [d16], ea16)
            return carry
        lax.fori_loop(0, EPW // L, step, 0)

        pltpu.sync_copy(eav, ea_h.at[pl.ds(base, EPW)])
        pltpu.sync_copy(degv, degp_h.at[pl.ds(wid * N, N)])

    return pl.kernel(
        body,
        out_type=(jax.ShapeDtypeStruct((EP,), F32),
                  jax.ShapeDtypeStruct((NW * N,), F32)),
        mesh=_sc_mesh(),
        scratch_types=[
            pltpu.VMEM((EPW,), I32), pltpu.VMEM((EPW,), I32),
            pltpu.VMEM((2 * NP,), F32), pltpu.VMEM((N,), F32),
            pltpu.VMEM((EPW,), F32),
        ],
        compiler_params=_SC_PARAMS,
        name="sc_affinity2",
    )


def _make_conv(N, NP, EP, CH, col_split):
    if col_split:
        EPS = EP // 16
    else:
        EPS = EP // NW
    STEPS = EPS // CH
    RPT = NP // 16
    ZR = 128
    NZ = RPT // ZR

    def body(src_h, dst_h, w_h, dis_h, feat_h, out_h,
             disv, s_i, d_i, e_v, gi, c1v, rows, zb, acc, sem):
        c = lax.axis_index("c")
        s = lax.axis_index("s")
        cNP = c * NP
        pltpu.sync_copy(dis_h, disv)

        def zrow(i, carry):
            for b in range(8):
                zb[i, pl.ds(b * L, L)] = jnp.full((L,), 0.0, F32)
            return carry
        lax.fori_loop(0, ZR, zrow, 0)
        r0 = s * RPT
        for j in range(NZ):
            pltpu.sync_copy(zb, acc.at[pl.ds(r0 + j * ZR, ZR)])
        plsc.subcore_barrier()

        def step(g, carry):
            if col_split:
                base = s * EPS + g * CH
            else:
                base = c * (EP // 2) + s * EPS + g * CH
            pltpu.sync_copy(src_h.at[pl.ds(base, CH)], s_i)
            pltpu.sync_copy(dst_h.at[pl.ds(base, CH)], d_i)
            pltpu.sync_copy(w_h.at[pl.ds(base, CH)], e_v)
            for b in range(CH // L):
                s16 = s_i[pl.ds(b * L, L)]
                d16 = d_i[pl.ds(b * L, L)]
                ew16 = e_v[pl.ds(b * L, L)]
                if col_split:
                    gi[pl.ds(b * L, L)] = s16 + cNP
                c1v[pl.ds(b * L, L)] = (plsc.load_gather(disv, [s16]) * ew16
                                        * plsc.load_gather(disv, [d16]))
            idx_ref = gi if col_split else s_i
            pltpu.async_copy(feat_h.at[idx_ref], rows, sem).wait()

            def scale(j, carry2):
                cj = plsc.load_gather(c1v, [jnp.broadcast_to(j, (L,)).astype(I32)])
                for b in range(8):
                    rows[j, pl.ds(b * L, L)] = rows[j, pl.ds(b * L, L)] * cj
                return carry2
            lax.fori_loop(0, CH, scale, 0)
            pltpu.sync_copy(rows, acc.at[d_i], add=True)
            return carry
        lax.fori_loop(0, STEPS, step, 0)
        plsc.subcore_barrier()

        for j in range(NZ):
            pltpu.sync_copy(acc.at[pl.ds(r0 + j * ZR, ZR)], zb)
            pltpu.sync_copy(zb, out_h.at[pl.ds(cNP + r0 + j * ZR, ZR)])

    return pl.kernel(
        body,
        out_type=jax.ShapeDtypeStruct((2 * NP, 128), F32),
        mesh=_sc_mesh(),
        scratch_types=[
            pltpu.VMEM((N,), F32),
            pltpu.VMEM((CH,), I32), pltpu.VMEM((CH,), I32),
            pltpu.VMEM((CH,), F32), pltpu.VMEM((CH,), I32),
            pltpu.VMEM((CH,), F32),
            pltpu.VMEM((CH, 128), F32),
            pltpu.VMEM((ZR, 128), F32),
            pltpu.VMEM_SHARED((NP, 128), F32),
            pltpu.SemaphoreType.DMA,
        ],
        compiler_params=_SC_PARAMS,
        name="sc_conv_col" if col_split else "sc_conv_edge",
    )


def _make_diff(EP2, CH):
    EPW = EP2 // NW
    STEPS = EPW // CH

    def body(d0_h, d1_h, outf_h, inp_h, i0, i1, r0v, r1v, sem0, sem1):
        c = lax.axis_index("c")
        s = lax.axis_index("s")
        wid = s * 2 + c

        def step(g, carry):
            base = wid * EPW + g * CH
            pltpu.sync_copy(d0_h.at[pl.ds(base, CH)], i0)
            pltpu.sync_copy(d1_h.at[pl.ds(base, CH)], i1)
            h0 = pltpu.async_copy(outf_h.at[i0], r0v, sem0)
            h1 = pltpu.async_copy(outf_h.at[i1], r1v, sem1)
            h0.wait()
            h1.wait()

            def diff(j, carry2):
                for b in range(8):
                    r0v[j, pl.ds(b * L, L)] = (r0v[j, pl.ds(b * L, L)]
                                               - r1v[j, pl.ds(b * L, L)])
                return carry2
            lax.fori_loop(0, CH, diff, 0)
            pltpu.sync_copy(r0v, inp_h.at[pl.ds(base, CH)])
            return carry
        lax.fori_loop(0, STEPS, step, 0)

    return pl.kernel(
        body,
        out_type=jax.ShapeDtypeStruct((EP2, 128), F32),
        mesh=_sc_mesh(),
        scratch_types=[
            pltpu.VMEM((CH,), I32), pltpu.VMEM((CH,), I32),
            pltpu.VMEM((CH, 128), F32), pltpu.VMEM((CH, 128), F32),
            pltpu.SemaphoreType.DMA, pltpu.SemaphoreType.DMA,
        ],
        compiler_params=_SC_PARAMS,
        name="sc_diff",
    )


def _round_up(v, m):
    return ((v + m - 1) // m) * m


def kernel(x, coords, edge_index, data_edge_index, Wa, ba, Wg, bg, Waf, baf,
           Wc1, bc1, Wm1, bm1, Wc2, bc2, Wf1, bf1, Wf2, bf2):
    N, D = x.shape
    E = edge_index.shape[1]
    E2 = data_edge_index.shape[1]
    EP = _round_up(E, 4096)
    EP2 = _round_up(E2, 4096)
    NP = _round_up(N, 2048)
    BN = 1024
    GN = NP // BN
    BM = 1024

    src = _pad_edges(edge_index[0], EP)
    dst = _pad_edges(edge_index[1], EP)
    d0 = _pad_edges(data_edge_index[0], EP2)
    d1 = _pad_edges(data_edge_index[1], EP2)
    xp = jnp.concatenate([x, jnp.zeros((NP - N, D), F32)])

    Wpa = jnp.stack([Wa[0, :D], Wa[0, D:]])
    Wgs = jnp.stack([Wg[0, :4], Wg[0, 4:]])
    b4 = jnp.stack([ba[0], 0.0, bg[0], 0.0]).reshape(4, 1).astype(F32)
    cst = jnp.concatenate([jnp.broadcast_to(Waf[0, 0], (L,)),
                           jnp.broadcast_to(Waf[0, 1], (L,)),
                           jnp.broadcast_to(baf[0], (L,))]).astype(F32)
    bc1r = bc1.reshape(2, 1, 128)
    Wc2a = Wc2[:, :128]
    Wc2b = Wc2[:, 128:]
    Wqc = jnp.stack([Wm1[0, :256], Wm1[0, 256:]], axis=1)
    Wqa = Wqc[:128]
    Wqb = Wqc[128:]
    bq = jnp.stack([bm1[0], 0.0]).reshape(1, 2).astype(F32)
    bf1r = bf1.reshape(1, 64)
    bf2r = bf2.reshape(1, 1)

    pg = pl.pallas_call(
        _proj_body,
        out_shape=jax.ShapeDtypeStruct((4, N), F32),
    )(x, coords, Wpa, Wgs, b4)
    pgf = pg.reshape(4 * N)

    hw1 = pl.pallas_call(
        _hw1_body,
        grid=(GN,),
        in_specs=[pl.BlockSpec((BN, D), lambda i: (i, 0)),
                  pl.BlockSpec((256, D), lambda i: (0, 0))],
        out_specs=pl.BlockSpec((2, BN, 128), lambda i: (0, i, 0)),
        out_shape=jax.ShapeDtypeStruct((2, NP, 128), F32),
    )(xp, Wc1)
    hw1f = hw1.reshape(2 * NP, 128)

    ew, degp = _make_affinity(N, E, EP)(src, dst, pgf, cst)

    dis, inv = pl.pallas_call(
        _deg_body,
        out_shape=(jax.ShapeDtypeStruct((1, N), F32),
                   jax.ShapeDtypeStruct((1, N), F32)),
    )(degp.reshape(NW, N))
    disf = dis.reshape(N)
    invc = jnp.concatenate([inv.reshape(N, 1), jnp.zeros((NP - N, 1), F32)])

    acc1 = _make_conv(N, NP, EP, 128, True)(src, dst, ew, disf, hw1f)
    acc1v = acc1.reshape(2, NP, 128)

    hw2, q = pl.pallas_call(
        _stage_e_body,
        grid=(GN,),
        in_specs=[pl.BlockSpec((2, BN, 128), lambda i: (0, i, 0)),
                  pl.BlockSpec((2, BN, 128), lambda i: (0, i, 0)),
                  pl.BlockSpec((BN, 1), lambda i: (i, 0)),
                  pl.BlockSpec((2, 1, 128), lambda i: (0, 0, 0)),
                  pl.BlockSpec((128, 128), lambda i: (0, 0)),
                  pl.BlockSpec((128, 128), lambda i: (0, 0)),
                  pl.BlockSpec((128, 2), lambda i: (0, 0)),
                  pl.BlockSpec((128, 2), lambda i: (0, 0)),
                  pl.BlockSpec((1, 2), lambda i: (0, 0))],
        out_specs=(pl.BlockSpec((BN, 128), lambda i: (i, 0)),
                   pl.BlockSpec((BN, 2), lambda i: (i, 0))),
        out_shape=(jax.ShapeDtypeStruct((NP, 128), F32),
                   jax.ShapeDtypeStruct((NP, 2), F32)),
    )(acc1v, hw1, invc, bc1r, Wc2a, Wc2b, Wqa, Wqb, bq)
    qf = q.reshape(2 * NP)

    ea2, degp2 = _make_affinity2(N, NP, E, EP)(src, dst, qf)

    dis2, inv2 = pl.pallas_call(
        _deg_body,
        out_shape=(jax.ShapeDtypeStruct((1, N), F32),
                   jax.ShapeDtypeStruct((1, N), F32)),
    )(degp2.reshape(NW, N))
    dis2f = dis2.reshape(N)
    inv2c = jnp.concatenate([inv2.reshape(N, 1), jnp.zeros((NP - N, 1), F32)])

    acc2 = _make_conv(N, NP, EP, 128, False)(src, dst, ea2, dis2f, hw2)
    acc2v = acc2.reshape(2, NP, 128)

    outF = pl.pallas_call(
        _combine_body,
        grid=(GN,),
        in_specs=[pl.BlockSpec((2, BN, 128), lambda i: (0, i, 0)),
                  pl.BlockSpec((BN, 128), lambda i: (i, 0)),
                  pl.BlockSpec((BN, 1), lambda i: (i, 0))],
        out_specs=pl.BlockSpec((BN, 128), lambda i: (i, 0)),
        out_shape=jax.ShapeDtypeStruct((NP, 128), F32),
    )(acc2v, hw2, inv2c)

    inp = _make_diff(EP2, 128)(d0, d1, outF)

    prob = pl.pallas_call(
        _mlp_body,
        grid=(EP2 // BM,),
        in_specs=[pl.BlockSpec((BM, 128), lambda i: (i, 0)),
                  pl.BlockSpec((64, 128), lambda i: (0, 0)),
                  pl.BlockSpec((1, 64), lambda i: (0, 0)),
                  pl.BlockSpec((1, 64), lambda i: (0, 0)),
                  pl.BlockSpec((1, 1), lambda i: (0, 0))],
        out_specs=pl.BlockSpec((BM, 1), lambda i: (i, 0)),
        out_shape=jax.ShapeDtypeStruct((EP2, 1), F32),
    )(inp, Wf1, bf1r, Wf2, bf2r)

    return prob[:E2]
